# baseline (speedup 1.0000x reference)
import functools

import jax
import jax.numpy as jnp
from jax import lax
from jax.experimental import pallas as pl
from jax.experimental.pallas import tpu as pltpu
from jax.experimental.pallas import tpu_sc as plsc

N_U = 25000
N_I = 25000
NN = N_U + N_I
G = 16
E = 800000 + NN
C = 1024
SUB = 128
NSUB = C // SUB
CPT = 54
EP = 16 * CPT * C
NBLK = EP // C
CPT_S = NBLK // 32
NB = 3
NPAD = 51200
RPT = NPAD // 16
PIECES_FULL = ((0, 1024), (1024, 1024), (2048, 1024), (3072, 128))
PIECES_LAST = ((0, 1024), (1024, 976))
NP2 = 50048
BATCH = 4096
Q = 3 * BATCH
Q4 = 4 * Q
QPT = Q4 // 32
KQ = QPT // SUB

_mesh = plsc.VectorSubcoreMesh(core_axis_name="c", subcore_axis_name="s")
_sc_params = pltpu.CompilerParams(use_tc_tiling_on_sc=False)


@functools.partial(
    pl.kernel,
    out_type=jax.ShapeDtypeStruct((2 * NN,), jnp.float32),
    mesh=_mesh,
    compiler_params=_sc_params,
    scratch_types=[
        pltpu.VMEM((NSUB, SUB), jnp.int32),
        pltpu.VMEM((C,), jnp.float32),
        pltpu.VMEM_SHARED((NPAD,), jnp.float32),
    ],
)
def _s_kernel(rows_hbm, delta_hbm, s_out, rbuf, dbuf, sacc):
    c = lax.axis_index("c")
    w = lax.axis_index("s")

    def zfill(k, cc):
        dbuf[pl.ds(k * 16, 16)] = jnp.zeros((16,), jnp.float32)
        return cc

    lax.fori_loop(0, C // 16, zfill, 0)
    for (off, sz) in PIECES_FULL:
        pltpu.sync_copy(dbuf.at[pl.ds(0, sz)],
                        sacc.at[pl.ds(w * RPT + off, sz)])
    plsc.subcore_barrier()

    def chunk(k, carry):
        blk = c * (NBLK // 2) + w * CPT_S + k
        pltpu.sync_copy(rows_hbm.at[blk], rbuf)
        pltpu.sync_copy(delta_hbm.at[pl.ds(blk * C, C)], dbuf)
        for j in range(NSUB):
            pltpu.sync_copy(dbuf.at[pl.ds(j * SUB, SUB)],
                            sacc.at[rbuf.at[j]], add=True)
        return carry

    lax.fori_loop(0, CPT_S, chunk, 0)
    plsc.subcore_barrier()

    @pl.when(w < 15)
    def _():
        for (off, sz) in PIECES_FULL:
            pltpu.sync_copy(sacc.at[pl.ds(w * RPT + off, sz)],
                            dbuf.at[pl.ds(0, sz)])
            pltpu.sync_copy(dbuf.at[pl.ds(0, sz)],
                            s_out.at[pl.ds(c * NN + w * RPT + off, sz)])

    @pl.when(w == 15)
    def _():
        for (off, sz) in PIECES_LAST:
            pltpu.sync_copy(sacc.at[pl.ds(15 * RPT + off, sz)],
                            dbuf.at[pl.ds(0, sz)])
            pltpu.sync_copy(dbuf.at[pl.ds(0, sz)],
                            s_out.at[pl.ds(c * NN + 15 * RPT + off, sz)])


@functools.partial(
    pl.kernel,
    out_type=jax.ShapeDtypeStruct((4 * NP2, G), jnp.float32),
    mesh=_mesh,
    compiler_params=_sc_params,
    scratch_types=[
        pltpu.VMEM((NSUB, SUB), jnp.int32),
        pltpu.VMEM((NSUB, SUB), jnp.int32),
        pltpu.VMEM((C,), jnp.float32),
        pltpu.VMEM((C, G), jnp.float32),
        pltpu.VMEM_SHARED((NPAD, G), jnp.float32),
        pltpu.SemaphoreType.DMA,
        pltpu.SemaphoreType.DMA,
    ],
)
def _spmm_kernel(ego_hbm, cols_hbm, rows_hbm, vals_hbm, out_hbm,
                 cbuf, rbuf, vbuf, gbuf, acc, semg, sems):
    semgs = (semg,) * NSUB
    c = lax.axis_index("c")
    w = lax.axis_index("s")

    def drain(sz_rows, dst, sem):
        pltpu.make_async_copy(ego_hbm.at[pl.ds(0, sz_rows)], dst, sem).wait()

    for p in range(2):
        q = 2 * c + p

        def zfill(e, cc):
            gbuf[e] = jnp.zeros((G,), jnp.float32)
            return cc

        lax.fori_loop(0, C, zfill, 0)
        for (off, sz) in PIECES_FULL:
            pltpu.sync_copy(gbuf.at[pl.ds(0, sz)],
                            acc.at[pl.ds(w * RPT + off, sz)])
        plsc.subcore_barrier()

        def chunk(g, carry):
            blk = w * CPT + g
            pltpu.sync_copy(cols_hbm.at[q * NBLK + blk], cbuf)
            pltpu.sync_copy(rows_hbm.at[blk], rbuf)
            pltpu.sync_copy(vals_hbm.at[pl.ds(blk * C, C)], vbuf)
            for j in range(NSUB):
                pltpu.async_copy(ego_hbm.at[cbuf.at[j]],
                                 gbuf.at[pl.ds(j * SUB, SUB)], semgs[j])
            drain(C, gbuf, semgs[0])

            def mul(b, cc):
                vv = vbuf[pl.ds(b * 16, 16)]
                for t in range(16):
                    e = b * 16 + t
                    gbuf[e] = gbuf[e] * vv[t]
                return cc

            lax.fori_loop(0, C // 16, mul, 0)
            for j in range(NSUB):
                pltpu.async_copy(gbuf.at[pl.ds(j * SUB, SUB)],
                                 acc.at[rbuf.at[j]], sems, add=True)
            drain(C, gbuf, sems)
            return carry

        lax.fori_loop(0, CPT, chunk, 0)
        plsc.subcore_barrier()

        @pl.when(w < 15)
        def _():
            for (off, sz) in PIECES_FULL:
                pltpu.sync_copy(acc.at[pl.ds(w * RPT + off, sz)],
                                gbuf.at[pl.ds(0, sz)])
                pltpu.sync_copy(gbuf.at[pl.ds(0, sz)],
                                out_hbm.at[pl.ds(q * NP2 + w * RPT + off, sz)])

        @pl.when(w == 15)
        def _():
            for (off, sz) in PIECES_LAST:
                pltpu.sync_copy(acc.at[pl.ds(15 * RPT + off, sz)],
                                gbuf.at[pl.ds(0, sz)])
                pltpu.sync_copy(gbuf.at[pl.ds(0, sz)],
                                out_hbm.at[pl.ds(q * NP2 + 15 * RPT + off, sz)])


@functools.partial(
    pl.kernel,
    out_type=jax.ShapeDtypeStruct((4 * Q4, G), jnp.float32),
    mesh=_mesh,
    compiler_params=_sc_params,
    scratch_types=[
        pltpu.VMEM((KQ, SUB), jnp.int32),
        pltpu.VMEM((SUB, G), jnp.float32),
    ],
)
def _gather_kernel(t0, t1, t2, t3, idx_hbm, out_hbm, ibuf, gb):
    c = lax.axis_index("c")
    w = lax.axis_index("s") * 2 + c
    pltpu.sync_copy(idx_hbm.at[w], ibuf)
    for m, tab in enumerate((t0, t1, t2, t3)):
        for k in range(KQ):
            pltpu.sync_copy(tab.at[ibuf.at[k]], gb)
            pltpu.sync_copy(
                gb, out_hbm.at[pl.ds(m * Q4 + w * QPT + k * SUB, SUB)])


_BR = 368
_RQ = NP2 // 8


def _dense_body(s0, s1, s2, s3, e0, e1, e2, e3, sx_ref,
                wg_ref, wb_ref, bx_ref, o0, o1, o2, o3, n0, n1, n2, n3):
    f32 = jnp.float32
    side = jnp.concatenate([s0[...], s1[...], s2[...], s3[...]], axis=1)
    ego = jnp.concatenate([e0[...], e1[...], e2[...], e3[...]], axis=1)
    sx = sx_ref[...]
    sxx = jnp.concatenate([sx, sx, sx, sx], axis=1)
    side_l = side - sxx * ego
    pre = (jnp.dot(side, wg_ref[...], preferred_element_type=f32)
           + jnp.dot(ego * side_l, wb_ref[...], preferred_element_type=f32)
           + bx_ref[...])
    en = jnp.where(pre >= 0, pre, 0.01 * pre)
    ii = lax.broadcasted_iota(jnp.int32, (512, 8), 0)
    bb = lax.broadcasted_iota(jnp.int32, (512, 8), 1)
    P = ((ii % 128) // 16 == bb).astype(f32)
    t8 = jnp.dot(jnp.abs(en), P, preferred_element_type=f32)
    tl = jnp.dot(t8, P.T, preferred_element_type=f32)
    nm = en * (1.0 / jnp.maximum(tl, 1e-12))
    for qq, (oe, on) in enumerate(((o0, n0), (o1, n1), (o2, n2), (o3, n3))):
        oe[...] = en[:, qq * 128:(qq + 1) * 128]
        on[...] = nm[:, qq * 128:(qq + 1) * 128]


def _dense_call(side_p, ego_p, s_expand, WgBD, WbBD, bx):
    grid = _RQ // _BR
    full = lambda shape: pl.BlockSpec(shape, lambda b: (0,) * len(shape))
    rspec = lambda qq: pl.BlockSpec((_BR, 128), lambda b, qq=qq: (qq * grid + b, 0))
    ospec = pl.BlockSpec((_BR, 128), lambda b: (b, 0))
    in_specs = ([rspec(qq) for qq in range(4)] * 2
                + [ospec, full((512, 512)), full((512, 512)), full((1, 512))])
    outs = pl.pallas_call(
        _dense_body,
        grid=(grid,),
        in_specs=in_specs,
        out_specs=[ospec] * 8,
        out_shape=[jax.ShapeDtypeStruct((_RQ, 128), jnp.float32)] * 8,
    )(side_p, side_p, side_p, side_p, ego_p, ego_p, ego_p, ego_p,
      s_expand, WgBD, WbBD, bx)
    oe = jnp.concatenate(outs[:4], axis=0)
    on = jnp.concatenate(outs[4:], axis=0)
    return oe, on


def _expand_weight(W):
    ii = jnp.arange(512)
    feat = 16 * (ii // 128) + ii % 16
    node = (ii % 128) // 16
    return W[feat[:, None], feat[None, :]] * (
        node[:, None] == node[None, :]).astype(jnp.float32)


def _expand_bias(b):
    return jnp.concatenate(
        [jnp.tile(b[:, 16 * qq:16 * (qq + 1)], (1, 8)) for qq in range(4)],
        axis=1)


_PB = BATCH // 8


def _loss_body(g_ref, out_ref):
    S = jnp.zeros((_PB, 128), jnp.float32)
    for pc in range(16):
        gg = g_ref[pc]
        S = S + gg[0:_PB] * (gg[_PB:2 * _PB] - gg[2 * _PB:3 * _PB])
    li = lax.broadcasted_iota(jnp.int32, (128, 8), 0)
    bi = lax.broadcasted_iota(jnp.int32, (128, 8), 1)
    P = (li // 16 == bi).astype(jnp.float32)
    du = jnp.dot(S, P, preferred_element_type=jnp.float32)
    ls = jnp.minimum(du, 0.0) - jnp.log1p(jnp.exp(-jnp.abs(du)))
    out_ref[0, 0] = -jnp.mean(ls)


def kernel(user_embedding, item_embedding, W_gc_0, b_gc_0, W_bi_0, b_bi_0,
           W_gc_1, b_gc_1, W_bi_1, b_bi_1, W_gc_2, b_gc_2, W_bi_2, b_bi_2,
           rows, cols, a_vals, l_vals, u, i, j):
    f32 = jnp.float32
    i32 = jnp.int32
    pad = EP - E
    rows_p = jnp.concatenate([rows, jnp.zeros((pad,), i32)])
    cols_p = jnp.concatenate([cols, jnp.zeros((pad,), i32)])
    vals_p = jnp.concatenate([a_vals, jnp.zeros((pad,), f32)])
    delta_p = jnp.concatenate([a_vals - l_vals, jnp.zeros((pad,), f32)])
    rows3d = rows_p.reshape(NBLK, NSUB, SUB)
    cols4 = jnp.concatenate([cols_p + qq * NP2 for qq in range(4)]).reshape(
        4 * NBLK, NSUB, SUB)

    ego0 = jnp.concatenate([user_embedding, item_embedding,
                            jnp.zeros((NP2 - NN, 64), f32)], axis=0)
    egoq = jnp.concatenate([ego0[:, qq * G:(qq + 1) * G] for qq in range(4)],
                           axis=0)
    egoq0 = egoq

    s2 = _s_kernel(rows3d, delta_p)
    s_sum = jnp.concatenate([s2[:NN] + s2[NN:], jnp.zeros((NP2 - NN,), f32)])
    s_expand = jnp.repeat(s_sum, G).reshape(_RQ, 128)

    norms = []
    for (Wg, bg, Wb, bb) in ((W_gc_0, b_gc_0, W_bi_0, b_bi_0),
                             (W_gc_1, b_gc_1, W_bi_1, b_bi_1),
                             (W_gc_2, b_gc_2, W_bi_2, b_bi_2)):
        side = _spmm_kernel(egoq, cols4, rows3d, vals_p)
        oe, on = _dense_call(side.reshape(4 * _RQ, 128),
                             egoq.reshape(4 * _RQ, 128),
                             s_expand, _expand_weight(Wg), _expand_weight(Wb),
                             _expand_bias(bg) + _expand_bias(bb))
        egoq = oe.reshape(4 * NP2, G)
        norms.append(on.reshape(4 * NP2, G))

    idx = jnp.concatenate([u, N_U + i, N_U + j])
    idx4 = jnp.concatenate([idx + qq * NP2 for qq in range(4)]).reshape(
        32, KQ, SUB)
    gathered = _gather_kernel(egoq0, norms[0], norms[1], norms[2], idx4)
    g4 = gathered.reshape(16, 3 * _PB, 128)

    loss = pl.pallas_call(
        _loss_body,
        out_shape=jax.ShapeDtypeStruct((1, 1), jnp.float32),
        in_specs=[pl.BlockSpec(memory_space=pltpu.VMEM)],
        out_specs=pl.BlockSpec(memory_space=pltpu.SMEM),
    )(g4)
    return loss[0, 0]


# speedup vs baseline: 7.2437x; 7.2437x over previous
import functools

import jax
import jax.numpy as jnp
from jax import lax
from jax.experimental import pallas as pl
from jax.experimental.pallas import tpu as pltpu
from jax.experimental.pallas import tpu_sc as plsc

N_U = 25000
N_I = 25000
NN = N_U + N_I
G = 16
E = 800000 + NN
C = 2048
SUB = 128
NSUB = C // SUB
CPT = 26
EP = 16 * CPT * C
NBLK = EP // C
CPT_S = NBLK // 32
NPAD = 51200
RPT = NPAD // 16
PIECES_FULL = ((0, 2048), (2048, 1152))
PIECES_LAST = ((0, 2000),)
BATCH = 4096
Q = 3 * BATCH
Q4 = 4 * Q
QPT = Q4 // 32
KQ = QPT // SUB

_mesh = plsc.VectorSubcoreMesh(core_axis_name="c", subcore_axis_name="s")
_sc_params = pltpu.CompilerParams(use_tc_tiling_on_sc=False)


@functools.partial(
    pl.kernel,
    out_type=jax.ShapeDtypeStruct((2 * NN,), jnp.float32),
    mesh=_mesh,
    compiler_params=_sc_params,
    scratch_types=[
        pltpu.VMEM((NSUB, SUB), jnp.int32),
        pltpu.VMEM((C,), jnp.float32),
        pltpu.VMEM_SHARED((NPAD,), jnp.float32),
    ],
)
def _s_kernel(rows_hbm, delta_hbm, s_out, rbuf, dbuf, sacc):
    c = lax.axis_index("c")
    w = lax.axis_index("s")

    def zfill(k, cc):
        dbuf[pl.ds(k * 16, 16)] = jnp.zeros((16,), jnp.float32)
        return cc

    lax.fori_loop(0, C // 16, zfill, 0)
    for (off, sz) in PIECES_FULL:
        pltpu.sync_copy(dbuf.at[pl.ds(0, sz)],
                        sacc.at[pl.ds(w * RPT + off, sz)])
    plsc.subcore_barrier()

    def chunk(k, carry):
        blk = c * (NBLK // 2) + w * CPT_S + k
        pltpu.sync_copy(rows_hbm.at[blk], rbuf)
        pltpu.sync_copy(delta_hbm.at[pl.ds(blk * C, C)], dbuf)
        for j in range(NSUB):
            pltpu.sync_copy(dbuf.at[pl.ds(j * SUB, SUB)],
                            sacc.at[rbuf.at[j]], add=True)
        return carry

    lax.fori_loop(0, CPT_S, chunk, 0)
    plsc.subcore_barrier()

    @pl.when(w < 15)
    def _():
        for (off, sz) in PIECES_FULL:
            pltpu.sync_copy(sacc.at[pl.ds(w * RPT + off, sz)],
                            dbuf.at[pl.ds(0, sz)])
            pltpu.sync_copy(dbuf.at[pl.ds(0, sz)],
                            s_out.at[pl.ds(c * NN + w * RPT + off, sz)])

    @pl.when(w == 15)
    def _():
        for (off, sz) in PIECES_LAST:
            pltpu.sync_copy(sacc.at[pl.ds(15 * RPT + off, sz)],
                            dbuf.at[pl.ds(0, sz)])
            pltpu.sync_copy(dbuf.at[pl.ds(0, sz)],
                            s_out.at[pl.ds(c * NN + 15 * RPT + off, sz)])


@functools.partial(
    pl.kernel,
    out_type=jax.ShapeDtypeStruct((4 * NN, G), jnp.float32),
    mesh=_mesh,
    compiler_params=_sc_params,
    scratch_types=[
        pltpu.VMEM((NSUB, SUB), jnp.int32),
        pltpu.VMEM((NSUB, SUB), jnp.int32),
        pltpu.VMEM((C,), jnp.float32),
        pltpu.VMEM((C, G), jnp.float32),
        pltpu.VMEM_SHARED((NPAD, G), jnp.float32),
        pltpu.SemaphoreType.DMA,
    ],
)
def _spmm_kernel(ego_hbm, cols_hbm, rows_hbm, vals_hbm, out_hbm,
                 cbuf, rbuf, vbuf, gbuf, acc, sem):
    c = lax.axis_index("c")
    w = lax.axis_index("s")

    for p in range(2):
        q = 2 * c + p

        def zfill(e, cc):
            gbuf[e] = jnp.zeros((G,), jnp.float32)
            return cc

        lax.fori_loop(0, C, zfill, 0)
        for (off, sz) in PIECES_FULL:
            pltpu.sync_copy(gbuf.at[pl.ds(0, sz)],
                            acc.at[pl.ds(w * RPT + off, sz)])
        plsc.subcore_barrier()

        def chunk(g, carry):
            blk = w * CPT + g
            pltpu.sync_copy(cols_hbm.at[q * NBLK + blk], cbuf)
            pltpu.sync_copy(rows_hbm.at[blk], rbuf)
            pltpu.sync_copy(vals_hbm.at[pl.ds(blk * C, C)], vbuf)
            descs = [
                pltpu.async_copy(ego_hbm.at[cbuf.at[j]],
                                 gbuf.at[pl.ds(j * SUB, SUB)], sem)
                for j in range(NSUB)
            ]
            for d in descs:
                d.wait()

            def mul(b, cc):
                vv = vbuf[pl.ds(b * 16, 16)]
                for t in range(16):
                    e = b * 16 + t
                    gbuf[e] = gbuf[e] * vv[t]
                return cc

            lax.fori_loop(0, C // 16, mul, 0)
            descs = [
                pltpu.async_copy(gbuf.at[pl.ds(j * SUB, SUB)],
                                 acc.at[rbuf.at[j]], sem, add=True)
                for j in range(NSUB)
            ]
            for d in descs:
                d.wait()
            return carry

        lax.fori_loop(0, CPT, chunk, 0)
        plsc.subcore_barrier()

        @pl.when(w < 15)
        def _():
            for (off, sz) in PIECES_FULL:
                pltpu.sync_copy(acc.at[pl.ds(w * RPT + off, sz)],
                                gbuf.at[pl.ds(0, sz)])
                pltpu.sync_copy(gbuf.at[pl.ds(0, sz)],
                                out_hbm.at[pl.ds(q * NN + w * RPT + off, sz)])

        @pl.when(w == 15)
        def _():
            for (off, sz) in PIECES_LAST:
                pltpu.sync_copy(acc.at[pl.ds(15 * RPT + off, sz)],
                                gbuf.at[pl.ds(0, sz)])
                pltpu.sync_copy(gbuf.at[pl.ds(0, sz)],
                                out_hbm.at[pl.ds(q * NN + 15 * RPT + off, sz)])


@functools.partial(
    pl.kernel,
    out_type=jax.ShapeDtypeStruct((4 * Q4, G), jnp.float32),
    mesh=_mesh,
    compiler_params=_sc_params,
    scratch_types=[
        pltpu.VMEM((KQ, SUB), jnp.int32),
        pltpu.VMEM((SUB, G), jnp.float32),
    ],
)
def _gather_kernel(t0, t1, t2, t3, idx_hbm, out_hbm, ibuf, gb):
    c = lax.axis_index("c")
    w = lax.axis_index("s") * 2 + c
    pltpu.sync_copy(idx_hbm.at[w], ibuf)
    for m, tab in enumerate((t0, t1, t2, t3)):
        for k in range(KQ):
            pltpu.sync_copy(tab.at[ibuf.at[k]], gb)
            pltpu.sync_copy(
                gb, out_hbm.at[pl.ds(m * Q4 + w * QPT + k * SUB, SUB)])


def _dense_body(side_ref, ego_ref, sc_ref, wg_ref, wb_ref, bg_ref, bb_ref,
                oe_ref, on_ref):
    scol = jnp.sum(sc_ref[...], axis=1, keepdims=True)
    ego = jnp.concatenate([ego_ref[qq] for qq in range(4)], axis=1)
    side = jnp.concatenate([side_ref[qq] for qq in range(4)], axis=1)
    side_l = side - scol * ego
    f32 = jnp.float32
    pre = (jnp.dot(side, wg_ref[...], preferred_element_type=f32)
           + jnp.dot(ego * side_l, wb_ref[...], preferred_element_type=f32)
           + bg_ref[...] + bb_ref[...])
    en = jnp.where(pre >= 0, pre, 0.01 * pre)
    t = jnp.sum(jnp.abs(en), axis=1, keepdims=True)
    nm = en * (1.0 / jnp.maximum(t, 1e-12))
    for qq in range(4):
        oe_ref[qq] = en[:, qq * G:(qq + 1) * G]
        on_ref[qq] = nm[:, qq * G:(qq + 1) * G]


_BN = 2000


def _dense_call(side4, ego4, s_cols, Wg, bg, Wb, bb):
    grid = NN // _BN
    full = lambda shape: pl.BlockSpec(shape, lambda b: (0,) * len(shape))
    qspec = pl.BlockSpec((4, _BN, G), lambda b: (0, b, 0))
    in_specs = [qspec, qspec, pl.BlockSpec((_BN, 2), lambda b: (b, 0)),
                full((64, 64)), full((64, 64)), full((1, 64)), full((1, 64))]
    oe, on = pl.pallas_call(
        _dense_body,
        grid=(grid,),
        in_specs=in_specs,
        out_specs=[qspec, qspec],
        out_shape=[jax.ShapeDtypeStruct((4, NN, G), jnp.float32)] * 2,
    )(side4, ego4, s_cols, Wg, Wb, bg, bb)
    return oe, on


_PB = BATCH // 8


def _loss_body(g_ref, out_ref):
    S = jnp.zeros((_PB, 128), jnp.float32)
    for pc in range(16):
        gg = g_ref[pc]
        S = S + gg[0:_PB] * (gg[_PB:2 * _PB] - gg[2 * _PB:3 * _PB])
    li = lax.broadcasted_iota(jnp.int32, (128, 8), 0)
    bi = lax.broadcasted_iota(jnp.int32, (128, 8), 1)
    P = (li // 16 == bi).astype(jnp.float32)
    du = jnp.dot(S, P, preferred_element_type=jnp.float32)
    ls = jnp.minimum(du, 0.0) - jnp.log1p(jnp.exp(-jnp.abs(du)))
    out_ref[0, 0] = -jnp.mean(ls)


def kernel(user_embedding, item_embedding, W_gc_0, b_gc_0, W_bi_0, b_bi_0,
           W_gc_1, b_gc_1, W_bi_1, b_bi_1, W_gc_2, b_gc_2, W_bi_2, b_bi_2,
           rows, cols, a_vals, l_vals, u, i, j):
    f32 = jnp.float32
    i32 = jnp.int32
    pad = EP - E
    rows_p = jnp.concatenate([rows, jnp.zeros((pad,), i32)])
    cols_p = jnp.concatenate([cols, jnp.zeros((pad,), i32)])
    vals_p = jnp.concatenate([a_vals, jnp.zeros((pad,), f32)])
    delta_p = jnp.concatenate([a_vals - l_vals, jnp.zeros((pad,), f32)])
    rows3d = rows_p.reshape(NBLK, NSUB, SUB)
    cols4 = jnp.concatenate([cols_p + qq * NN for qq in range(4)]).reshape(
        4 * NBLK, NSUB, SUB)

    ego0 = jnp.concatenate([user_embedding, item_embedding], axis=0)
    egoq = jnp.concatenate([ego0[:, qq * G:(qq + 1) * G] for qq in range(4)],
                           axis=0)
    egoq0 = egoq

    s2 = _s_kernel(rows3d, delta_p)
    s_cols = jnp.stack([s2[:NN], s2[NN:]], axis=1)

    norms = []
    for (Wg, bg, Wb, bb) in ((W_gc_0, b_gc_0, W_bi_0, b_bi_0),
                             (W_gc_1, b_gc_1, W_bi_1, b_bi_1),
                             (W_gc_2, b_gc_2, W_bi_2, b_bi_2)):
        side = _spmm_kernel(egoq, cols4, rows3d, vals_p)
        oe, on = _dense_call(side.reshape(4, NN, G),
                             egoq.reshape(4, NN, G),
                             s_cols, Wg, bg, Wb, bb)
        egoq = oe.reshape(4 * NN, G)
        norms.append(on.reshape(4 * NN, G))

    idx = jnp.concatenate([u, N_U + i, N_U + j])
    idx4 = jnp.concatenate([idx + qq * NN for qq in range(4)]).reshape(
        32, KQ, SUB)
    gathered = _gather_kernel(egoq0, norms[0], norms[1], norms[2], idx4)
    g4 = gathered.reshape(16, 3 * _PB, 128)

    loss = pl.pallas_call(
        _loss_body,
        out_shape=jax.ShapeDtypeStruct((1, 1), jnp.float32),
        in_specs=[pl.BlockSpec(memory_space=pltpu.VMEM)],
        out_specs=pl.BlockSpec(memory_space=pltpu.SMEM),
    )(g4)
    return loss[0, 0]


# speedup vs baseline: 7.5402x; 1.0409x over previous
import functools

import jax
import jax.numpy as jnp
from jax import lax
from jax.experimental import pallas as pl
from jax.experimental.pallas import tpu as pltpu
from jax.experimental.pallas import tpu_sc as plsc

N_U = 25000
N_I = 25000
NN = N_U + N_I
G = 16
E = 800000 + NN
C = 2048
SUB = 128
NSUB = C // SUB
CPT = 26
EP = 16 * CPT * C
NBLK = EP // C
CPT_S = NBLK // 32
C2 = 4096
NSUB2 = C2 // SUB
CPT2 = EP // (16 * C2)
NBLK2 = EP // C2
NPAD = 51200
RPT = NPAD // 16
PIECES_FULL = ((0, 2048), (2048, 1152))
PIECES_LAST = ((0, 2000),)
PIECES_FULL2 = ((0, 3200),)
BATCH = 4096
Q = 3 * BATCH
Q4 = 4 * Q
QPT = Q4 // 32
KQ = QPT // SUB

_mesh = plsc.VectorSubcoreMesh(core_axis_name="c", subcore_axis_name="s")
_sc_params = pltpu.CompilerParams(use_tc_tiling_on_sc=False)


@functools.partial(
    pl.kernel,
    out_type=jax.ShapeDtypeStruct((2 * NN,), jnp.float32),
    mesh=_mesh,
    compiler_params=_sc_params,
    scratch_types=[
        pltpu.VMEM((NSUB, SUB), jnp.int32),
        pltpu.VMEM((C,), jnp.float32),
        pltpu.VMEM_SHARED((NPAD,), jnp.float32),
    ],
)
def _s_kernel(rows_hbm, delta_hbm, s_out, rbuf, dbuf, sacc):
    c = lax.axis_index("c")
    w = lax.axis_index("s")

    def zfill(k, cc):
        dbuf[pl.ds(k * 16, 16)] = jnp.zeros((16,), jnp.float32)
        return cc

    lax.fori_loop(0, C // 16, zfill, 0)
    for (off, sz) in PIECES_FULL:
        pltpu.sync_copy(dbuf.at[pl.ds(0, sz)],
                        sacc.at[pl.ds(w * RPT + off, sz)])
    plsc.subcore_barrier()

    def chunk(k, carry):
        blk = c * (NBLK // 2) + w * CPT_S + k
        pltpu.sync_copy(rows_hbm.at[blk], rbuf)
        pltpu.sync_copy(delta_hbm.at[pl.ds(blk * C, C)], dbuf)
        for j in range(NSUB):
            pltpu.sync_copy(dbuf.at[pl.ds(j * SUB, SUB)],
                            sacc.at[rbuf.at[j]], add=True)
        return carry

    lax.fori_loop(0, CPT_S, chunk, 0)
    plsc.subcore_barrier()

    @pl.when(w < 15)
    def _():
        for (off, sz) in PIECES_FULL:
            pltpu.sync_copy(sacc.at[pl.ds(w * RPT + off, sz)],
                            dbuf.at[pl.ds(0, sz)])
            pltpu.sync_copy(dbuf.at[pl.ds(0, sz)],
                            s_out.at[pl.ds(c * NN + w * RPT + off, sz)])

    @pl.when(w == 15)
    def _():
        for (off, sz) in PIECES_LAST:
            pltpu.sync_copy(sacc.at[pl.ds(15 * RPT + off, sz)],
                            dbuf.at[pl.ds(0, sz)])
            pltpu.sync_copy(dbuf.at[pl.ds(0, sz)],
                            s_out.at[pl.ds(c * NN + 15 * RPT + off, sz)])


@functools.partial(
    pl.kernel,
    out_type=jax.ShapeDtypeStruct((4 * NN, G), jnp.float32),
    mesh=_mesh,
    compiler_params=_sc_params,
    scratch_types=[
        pltpu.VMEM((NSUB2, SUB), jnp.int32),
        pltpu.VMEM((NSUB2, SUB), jnp.int32),
        pltpu.VMEM((C2,), jnp.float32),
        pltpu.VMEM((C2, G), jnp.float32),
        pltpu.VMEM_SHARED((NPAD, G), jnp.float32),
        pltpu.SemaphoreType.DMA,
    ],
)
def _spmm_kernel(ego_hbm, cols_hbm, rows_hbm, vals_hbm, out_hbm,
                 cbuf, rbuf, vbuf, gbuf, acc, sem):
    c = lax.axis_index("c")
    w = lax.axis_index("s")

    for p in range(2):
        q = 2 * c + p

        def zfill(e, cc):
            gbuf[e] = jnp.zeros((G,), jnp.float32)
            return cc

        lax.fori_loop(0, C2, zfill, 0)
        for (off, sz) in PIECES_FULL2:
            pltpu.sync_copy(gbuf.at[pl.ds(0, sz)],
                            acc.at[pl.ds(w * RPT + off, sz)])
        plsc.subcore_barrier()

        def chunk(g, carry):
            blk = w * CPT2 + g
            pltpu.sync_copy(cols_hbm.at[q * NBLK2 + blk], cbuf)
            pltpu.sync_copy(rows_hbm.at[blk], rbuf)
            pltpu.sync_copy(vals_hbm.at[pl.ds(blk * C2, C2)], vbuf)
            descs = [
                pltpu.async_copy(ego_hbm.at[cbuf.at[j]],
                                 gbuf.at[pl.ds(j * SUB, SUB)], sem)
                for j in range(NSUB2)
            ]
            for d in descs:
                d.wait()

            def mul(b, cc):
                vv = vbuf[pl.ds(b * 16, 16)]
                for t in range(16):
                    e = b * 16 + t
                    gbuf[e] = gbuf[e] * vv[t]
                return cc

            lax.fori_loop(0, C2 // 16, mul, 0)
            descs = [
                pltpu.async_copy(gbuf.at[pl.ds(j * SUB, SUB)],
                                 acc.at[rbuf.at[j]], sem, add=True)
                for j in range(NSUB2)
            ]
            for d in descs:
                d.wait()
            return carry

        lax.fori_loop(0, CPT2, chunk, 0)
        plsc.subcore_barrier()

        @pl.when(w < 15)
        def _():
            for (off, sz) in PIECES_FULL2:
                pltpu.sync_copy(acc.at[pl.ds(w * RPT + off, sz)],
                                gbuf.at[pl.ds(0, sz)])
                pltpu.sync_copy(gbuf.at[pl.ds(0, sz)],
                                out_hbm.at[pl.ds(q * NN + w * RPT + off, sz)])

        @pl.when(w == 15)
        def _():
            for (off, sz) in PIECES_LAST:
                pltpu.sync_copy(acc.at[pl.ds(15 * RPT + off, sz)],
                                gbuf.at[pl.ds(0, sz)])
                pltpu.sync_copy(gbuf.at[pl.ds(0, sz)],
                                out_hbm.at[pl.ds(q * NN + 15 * RPT + off, sz)])


@functools.partial(
    pl.kernel,
    out_type=jax.ShapeDtypeStruct((4 * Q4, G), jnp.float32),
    mesh=_mesh,
    compiler_params=_sc_params,
    scratch_types=[
        pltpu.VMEM((KQ, SUB), jnp.int32),
        pltpu.VMEM((SUB, G), jnp.float32),
    ],
)
def _gather_kernel(t0, t1, t2, t3, idx_hbm, out_hbm, ibuf, gb):
    c = lax.axis_index("c")
    w = lax.axis_index("s") * 2 + c
    pltpu.sync_copy(idx_hbm.at[w], ibuf)
    for m, tab in enumerate((t0, t1, t2, t3)):
        for k in range(KQ):
            pltpu.sync_copy(tab.at[ibuf.at[k]], gb)
            pltpu.sync_copy(
                gb, out_hbm.at[pl.ds(m * Q4 + w * QPT + k * SUB, SUB)])


def _dense_body(side_ref, ego_ref, sc_ref, wg_ref, wb_ref, bg_ref, bb_ref,
                oe_ref, on_ref):
    scol = jnp.sum(sc_ref[...], axis=1, keepdims=True)
    ego = jnp.concatenate([ego_ref[qq] for qq in range(4)], axis=1)
    side = jnp.concatenate([side_ref[qq] for qq in range(4)], axis=1)
    side_l = side - scol * ego
    f32 = jnp.float32
    pre = (jnp.dot(side, wg_ref[...], preferred_element_type=f32)
           + jnp.dot(ego * side_l, wb_ref[...], preferred_element_type=f32)
           + bg_ref[...] + bb_ref[...])
    en = jnp.where(pre >= 0, pre, 0.01 * pre)
    t = jnp.sum(jnp.abs(en), axis=1, keepdims=True)
    nm = en * (1.0 / jnp.maximum(t, 1e-12))
    for qq in range(4):
        oe_ref[qq] = en[:, qq * G:(qq + 1) * G]
        on_ref[qq] = nm[:, qq * G:(qq + 1) * G]


_BN = 2000


def _dense_call(side4, ego4, s_cols, Wg, bg, Wb, bb):
    grid = NN // _BN
    full = lambda shape: pl.BlockSpec(shape, lambda b: (0,) * len(shape))
    qspec = pl.BlockSpec((4, _BN, G), lambda b: (0, b, 0))
    in_specs = [qspec, qspec, pl.BlockSpec((_BN, 2), lambda b: (b, 0)),
                full((64, 64)), full((64, 64)), full((1, 64)), full((1, 64))]
    oe, on = pl.pallas_call(
        _dense_body,
        grid=(grid,),
        in_specs=in_specs,
        out_specs=[qspec, qspec],
        out_shape=[jax.ShapeDtypeStruct((4, NN, G), jnp.float32)] * 2,
    )(side4, ego4, s_cols, Wg, Wb, bg, bb)
    return oe, on


_PB = BATCH // 8


def _loss_body(g_ref, out_ref):
    S = jnp.zeros((_PB, 128), jnp.float32)
    for pc in range(16):
        gg = g_ref[pc]
        S = S + gg[0:_PB] * (gg[_PB:2 * _PB] - gg[2 * _PB:3 * _PB])
    li = lax.broadcasted_iota(jnp.int32, (128, 8), 0)
    bi = lax.broadcasted_iota(jnp.int32, (128, 8), 1)
    P = (li // 16 == bi).astype(jnp.float32)
    du = jnp.dot(S, P, preferred_element_type=jnp.float32)
    ls = jnp.minimum(du, 0.0) - jnp.log1p(jnp.exp(-jnp.abs(du)))
    out_ref[0, 0] = -jnp.mean(ls)


def kernel(user_embedding, item_embedding, W_gc_0, b_gc_0, W_bi_0, b_bi_0,
           W_gc_1, b_gc_1, W_bi_1, b_bi_1, W_gc_2, b_gc_2, W_bi_2, b_bi_2,
           rows, cols, a_vals, l_vals, u, i, j):
    f32 = jnp.float32
    i32 = jnp.int32
    pad = EP - E
    rows_p = jnp.concatenate([rows, jnp.zeros((pad,), i32)])
    cols_p = jnp.concatenate([cols, jnp.zeros((pad,), i32)])
    vals_p = jnp.concatenate([a_vals, jnp.zeros((pad,), f32)])
    delta_p = jnp.concatenate([a_vals - l_vals, jnp.zeros((pad,), f32)])
    rows3d = rows_p.reshape(NBLK, NSUB, SUB)
    rows3d2 = rows_p.reshape(NBLK2, NSUB2, SUB)
    cols4 = jnp.concatenate([cols_p + qq * NN for qq in range(4)]).reshape(
        4 * NBLK2, NSUB2, SUB)

    ego0 = jnp.concatenate([user_embedding, item_embedding], axis=0)
    egoq = jnp.concatenate([ego0[:, qq * G:(qq + 1) * G] for qq in range(4)],
                           axis=0)
    egoq0 = egoq

    s2 = _s_kernel(rows3d, delta_p)
    s_cols = jnp.stack([s2[:NN], s2[NN:]], axis=1)

    norms = []
    for (Wg, bg, Wb, bb) in ((W_gc_0, b_gc_0, W_bi_0, b_bi_0),
                             (W_gc_1, b_gc_1, W_bi_1, b_bi_1),
                             (W_gc_2, b_gc_2, W_bi_2, b_bi_2)):
        side = _spmm_kernel(egoq, cols4, rows3d2, vals_p)
        oe, on = _dense_call(side.reshape(4, NN, G),
                             egoq.reshape(4, NN, G),
                             s_cols, Wg, bg, Wb, bb)
        egoq = oe.reshape(4 * NN, G)
        norms.append(on.reshape(4 * NN, G))

    idx = jnp.concatenate([u, N_U + i, N_U + j])
    idx4 = jnp.concatenate([idx + qq * NN for qq in range(4)]).reshape(
        32, KQ, SUB)
    gathered = _gather_kernel(egoq0, norms[0], norms[1], norms[2], idx4)
    g4 = gathered.reshape(16, 3 * _PB, 128)

    loss = pl.pallas_call(
        _loss_body,
        out_shape=jax.ShapeDtypeStruct((1, 1), jnp.float32),
        in_specs=[pl.BlockSpec(memory_space=pltpu.VMEM)],
        out_specs=pl.BlockSpec(memory_space=pltpu.SMEM),
    )(g4)
    return loss[0, 0]
